# baseline (device time: 51344 ns/iter reference)
import jax
import jax.numpy as jnp
from jax import lax
from jax.experimental import pallas as pl
from jax.experimental.pallas import tpu as pltpu

K = 16
NEG = float("-inf")


def kernel(x):
    m, n_loc = x.shape

    def body(x_ref, out_ref, work_ref, mine_ref, peer_ref, send_sem, recv_sem):
        my_x = lax.axis_index("x")
        my_y = lax.axis_index("y")
        peer = (1 - my_x, my_y)

        barrier_sem = pltpu.get_barrier_semaphore()
        pl.semaphore_signal(
            barrier_sem, inc=1, device_id=peer,
            device_id_type=pl.DeviceIdType.MESH,
        )
        pl.semaphore_wait(barrier_sem, 1)

        work_ref[...] = x_ref[...].astype(jnp.bfloat16)
        iota = lax.broadcasted_iota(jnp.int32, (m, n_loc), 1)
        for k in range(K):
            vals = work_ref[...].astype(jnp.float32)
            mx = jnp.max(vals, axis=1, keepdims=True)
            mine_ref[:, k : k + 1] = mx.astype(jnp.bfloat16)
            idx = jnp.min(
                jnp.where(vals == mx, iota, n_loc), axis=1, keepdims=True
            )
            work_ref[...] = jnp.where(iota == idx, NEG, vals).astype(
                jnp.bfloat16
            )

        rdma = pltpu.make_async_remote_copy(
            src_ref=mine_ref,
            dst_ref=peer_ref,
            send_sem=send_sem,
            recv_sem=recv_sem,
            device_id=peer,
            device_id_type=pl.DeviceIdType.MESH,
        )
        rdma.start()
        rdma.wait()

        cand = jnp.concatenate(
            [mine_ref[...], peer_ref[...]], axis=1
        ).astype(jnp.float32)
        iota2 = lax.broadcasted_iota(jnp.int32, (m, 2 * K), 1)
        for k in range(K):
            mx = jnp.max(cand, axis=1, keepdims=True)
            out_ref[:, k : k + 1] = mx
            idx = jnp.min(
                jnp.where(cand == mx, iota2, 2 * K), axis=1, keepdims=True
            )
            cand = jnp.where(iota2 == idx, NEG, cand)

    return pl.pallas_call(
        body,
        out_shape=jax.ShapeDtypeStruct((m, K), jnp.float32),
        in_specs=[pl.BlockSpec(memory_space=pltpu.VMEM)],
        out_specs=pl.BlockSpec(memory_space=pltpu.VMEM),
        scratch_shapes=[
            pltpu.VMEM((m, n_loc), jnp.bfloat16),
            pltpu.VMEM((m, K), jnp.bfloat16),
            pltpu.VMEM((m, K), jnp.bfloat16),
            pltpu.SemaphoreType.DMA,
            pltpu.SemaphoreType.DMA,
        ],
        compiler_params=pltpu.CompilerParams(collective_id=0),
    )(x)


# device time: 25192 ns/iter; 2.0381x vs baseline; 2.0381x over previous
import jax
import jax.numpy as jnp
from jax import lax
from jax.experimental import pallas as pl
from jax.experimental.pallas import tpu as pltpu

K = 16
NEG = float("-inf")


def kernel(x):
    m, n_loc = x.shape

    def body(x_ref, out_ref, mine_ref, peer_ref, send_sem, recv_sem):
        my_x = lax.axis_index("x")
        my_y = lax.axis_index("y")
        peer = (1 - my_x, my_y)

        barrier_sem = pltpu.get_barrier_semaphore()
        pl.semaphore_signal(
            barrier_sem, inc=1, device_id=peer,
            device_id_type=pl.DeviceIdType.MESH,
        )
        pl.semaphore_wait(barrier_sem, 1)

        vals = x_ref[...]
        for k in range(K):
            mx = jnp.max(vals, axis=1, keepdims=True)
            mine_ref[:, k : k + 1] = mx
            if k < K - 1:
                vals = jnp.where(vals == mx, NEG, vals)

        rdma = pltpu.make_async_remote_copy(
            src_ref=mine_ref,
            dst_ref=peer_ref,
            send_sem=send_sem,
            recv_sem=recv_sem,
            device_id=peer,
            device_id_type=pl.DeviceIdType.MESH,
        )
        rdma.start()
        rdma.wait()

        cand = jnp.concatenate([mine_ref[...], peer_ref[...]], axis=1)
        for k in range(K):
            mx = jnp.max(cand, axis=1, keepdims=True)
            out_ref[:, k : k + 1] = mx
            if k < K - 1:
                cand = jnp.where(cand == mx, NEG, cand)

    return pl.pallas_call(
        body,
        out_shape=jax.ShapeDtypeStruct((m, K), jnp.float32),
        in_specs=[pl.BlockSpec(memory_space=pltpu.VMEM)],
        out_specs=pl.BlockSpec(memory_space=pltpu.VMEM),
        scratch_shapes=[
            pltpu.VMEM((m, K), jnp.float32),
            pltpu.VMEM((m, K), jnp.float32),
            pltpu.SemaphoreType.DMA,
            pltpu.SemaphoreType.DMA,
        ],
        compiler_params=pltpu.CompilerParams(collective_id=0),
    )(x)
